# Initial kernel scaffold; baseline (speedup 1.0000x reference)
import jax
import jax.numpy as jnp
from jax.experimental import pallas as pl


def kernel(inputs, embedding_table):
    raise NotImplementedError("write your pallas kernel here")


# speedup vs baseline: 1.7981x; 1.7981x over previous
import functools

import jax
import jax.numpy as jnp
from jax import lax
from jax.experimental import pallas as pl
from jax.experimental.pallas import tpu as pltpu
from jax.experimental.pallas import tpu_sc as plsc

DIM = 64

_info = plsc.get_sparse_core_info()
_NC, _NS = _info.num_cores, _info.num_subcores
_NW = _NC * _NS


@functools.cache
def _make(B: int, CH: int):
    assert B % (_NW * CH) == 0
    b_per_w = B // _NW
    n_ch = b_per_w // CH
    mesh = plsc.VectorSubcoreMesh(core_axis_name="c", subcore_axis_name="s")

    @functools.partial(
        pl.kernel,
        out_type=jax.ShapeDtypeStruct((B, DIM), jnp.float32),
        mesh=mesh,
        scratch_types=[
            pltpu.VMEM((CH,), jnp.int32),
            pltpu.VMEM((CH, DIM), jnp.float32),
            pltpu.SemaphoreType.DMA,
        ],
        compiler_params=pltpu.CompilerParams(use_tc_tiling_on_sc=False),
    )
    def gather_kernel(idx_hbm, table_hbm, out_hbm, idx_v, rows_v, sem):
        wid = lax.axis_index("s") * _NC + lax.axis_index("c")
        base = wid * b_per_w

        @pl.loop(0, n_ch)
        def _(g):
            off = base + g * CH
            pltpu.sync_copy(idx_hbm.at[pl.ds(off, CH)], idx_v)
            pltpu.async_copy(table_hbm.at[idx_v], rows_v, sem).wait()
            pltpu.sync_copy(rows_v, out_hbm.at[pl.ds(off, CH)])

    return gather_kernel


def kernel(inputs, embedding_table):
    s0, s1 = inputs.shape
    B = s0 * s1
    idx = inputs.reshape(B).astype(jnp.int32)
    out = _make(B, 512)(idx, embedding_table)
    return out.reshape(s0, s1, DIM)


# speedup vs baseline: 1.8751x; 1.0428x over previous
import functools

import jax
import jax.numpy as jnp
from jax import lax
from jax.experimental import pallas as pl
from jax.experimental.pallas import tpu as pltpu
from jax.experimental.pallas import tpu_sc as plsc

DIM = 64

_info = plsc.get_sparse_core_info()
_NC, _NS = _info.num_cores, _info.num_subcores
_NW = _NC * _NS


@functools.cache
def _make(B: int, CH: int, D: int):
    assert B % (_NW * CH) == 0
    b_per_w = B // _NW
    n_ch = b_per_w // CH
    assert n_ch % D == 0 and n_ch >= 2 * D
    mesh = plsc.VectorSubcoreMesh(core_axis_name="c", subcore_axis_name="s")

    @functools.partial(
        pl.kernel,
        out_type=jax.ShapeDtypeStruct((B, DIM), jnp.float32),
        mesh=mesh,
        scratch_types=[
            pltpu.VMEM((b_per_w,), jnp.int32),
            [pltpu.VMEM((CH, DIM), jnp.float32) for _ in range(D)],
            [pltpu.SemaphoreType.DMA for _ in range(D)],
            [pltpu.SemaphoreType.DMA for _ in range(D)],
        ],
        compiler_params=pltpu.CompilerParams(use_tc_tiling_on_sc=False),
    )
    def gather_kernel(idx_hbm, table_hbm, out_hbm, idx_v, rows, sg, sw):
        wid = lax.axis_index("s") * _NC + lax.axis_index("c")
        base = wid * b_per_w
        pltpu.sync_copy(idx_hbm.at[pl.ds(base, b_per_w)], idx_v)

        def gather(tc, b):
            return pltpu.async_copy(
                table_hbm.at[idx_v.at[pl.ds(tc * CH, CH)]], rows[b], sg[b])

        def gather_wait(tc, b):
            pltpu.make_async_copy(
                table_hbm.at[idx_v.at[pl.ds(tc * CH, CH)]], rows[b], sg[b]).wait()

        def write(tc, b):
            return pltpu.async_copy(
                rows[b], out_hbm.at[pl.ds(base + tc * CH, CH)], sw[b])

        def write_wait(tc, b):
            pltpu.make_async_copy(
                rows[b], out_hbm.at[pl.ds(base + tc * CH, CH)], sw[b]).wait()

        for b in range(D):
            gather(b, b)

        @pl.loop(0, n_ch - D, step=D)
        def _(t):
            for b in range(D):
                tc = t + b
                gather_wait(tc, b)
                write(tc, b)
                write_wait(tc, b)
                gather(tc + D, b)

        for b in range(D):
            tc = n_ch - D + b
            gather_wait(tc, b)
            write(tc, b)
        for b in range(D):
            write_wait(n_ch - D + b, b)

    return gather_kernel


def kernel(inputs, embedding_table):
    s0, s1 = inputs.shape
    B = s0 * s1
    idx = inputs.reshape(B).astype(jnp.int32)
    out = _make(B, 256, 4)(idx, embedding_table)
    return out.reshape(s0, s1, DIM)
